# baseline (speedup 1.0000x reference)
import functools

import jax
import jax.numpy as jnp
from jax import lax
from jax.experimental import pallas as pl
from jax.experimental.pallas import tpu as pltpu
from jax.experimental.pallas import tpu_sc as plsc

VOCAB = 1000000
EMBED_DIM = 32
BATCH = 4096
SEQ = 200

B = BATCH * SEQ
NC = 2
NS = 16
NW = NC * NS
NBB = BATCH // 128
GROUPS = B // 128
G_PER_W = GROUPS // NW
GPC = 5
CHUNK_ROWS = 128 * GPC
CHUNKS = G_PER_W // GPC
NBUF = 2


def _emb_body(table_hbm, idx_hbm, out_hbm, idx_v, rows_v, trans_v,
              gsem0, gsem1, wsem0, wsem1):
    wid = lax.axis_index("s") * NC + lax.axis_index("c")
    g_base = wid * G_PER_W

    s0 = g_base // NBB
    for k in range(8):
        sk = jnp.minimum(s0 + k, SEQ - 1)
        pltpu.sync_copy(idx_hbm.at[sk], idx_v.at[k])

    gsems = (gsem0, gsem1)
    wsems = (wsem0, wsem1)

    iotas = [lax.iota(jnp.int32, 16) + (16 * j) for j in range(8)]

    def pair_body(p, carry):
        gathers = []
        groups = []
        for b in range(NBUF):
            c = p * NBUF + b
            for gg in range(GPC):
                g = g_base + c * GPC + gg
                s = g // NBB
                bb = g % NBB
                groups.append((g, s, bb))
                gathers.append(pltpu.async_copy(
                    table_hbm.at[idx_v.at[s - s0, pl.ds(bb * 128, 128)]],
                    rows_v.at[b, pl.ds(gg * 128, 128), :],
                    gsems[b]))
        writes = []
        for b in range(NBUF):
            for gg in range(GPC):
                gathers[b * GPC + gg].wait()
            for gg in range(GPC):
                src = rows_v.at[b]
                tdst = trans_v.at[b, gg]
                rows16 = [iotas[j] + (gg * 128) for j in range(8)]

                @plsc.parallel_loop(0, EMBED_DIM, unroll=4)
                def _transpose_e(e0):
                    colv = lax.bitwise_and(iotas[0] + e0, 31)
                    elv = lax.bitwise_and(colv, 7)
                    rv = lax.shift_right_logical(colv, 3)
                    for j in range(8):
                        vec = plsc.load_gather(src, [rows16[j], colv])
                        plsc.store_scatter(tdst, [rv, elv, iotas[j]], vec)

                _, s, bb = groups[b * GPC + gg]
                writes.append(pltpu.async_copy(
                    trans_v.at[b, gg],
                    out_hbm.at[s, :, bb],
                    wsems[b]))
        for w in writes:
            w.wait()
        return carry

    lax.fori_loop(0, CHUNKS // NBUF, pair_body, 0)


_gather_call = pl.kernel(
    _emb_body,
    out_type=jax.ShapeDtypeStruct((SEQ, 4, NBB, 8, 128), jnp.float32),
    name="emb_gather",
    mesh=plsc.VectorSubcoreMesh(core_axis_name="c", subcore_axis_name="s"),
    compiler_params=pltpu.CompilerParams(use_tc_tiling_on_sc=False,
                                         needs_layout_passes=False),
    scratch_types=[
        pltpu.VMEM((8, BATCH), jnp.int32),
        pltpu.VMEM((NBUF, CHUNK_ROWS, EMBED_DIM), jnp.float32),
        pltpu.VMEM((NBUF, GPC, 4, 8, 128), jnp.float32),
        pltpu.SemaphoreType.DMA,
        pltpu.SemaphoreType.DMA,
        pltpu.SemaphoreType.DMA,
        pltpu.SemaphoreType.DMA,
    ],
)


def kernel(x, table):
    out5 = _gather_call(table, x.T.astype(jnp.int32))
    return out5.transpose(2, 4, 0, 1, 3).reshape(BATCH, SEQ, EMBED_DIM)


# speedup vs baseline: 1.0004x; 1.0004x over previous
import functools

import jax
import jax.numpy as jnp
from jax import lax
from jax.experimental import pallas as pl
from jax.experimental.pallas import tpu as pltpu
from jax.experimental.pallas import tpu_sc as plsc

VOCAB = 1000000
EMBED_DIM = 32
BATCH = 4096
SEQ = 200

B = BATCH * SEQ
NC = 2
NS = 16
NW = NC * NS
NBB = BATCH // 128
GROUPS = B // 128
G_PER_W = GROUPS // NW
GPC = 5
CHUNK_ROWS = 128 * GPC
CHUNKS = G_PER_W // GPC
NBUF = 2


def _x_relayout_body(t_ref, o_ref):
    o_ref[0] = t_ref[...]


_x_relayout = pl.pallas_call(
    _x_relayout_body,
    out_shape=jax.ShapeDtypeStruct((SEQ // 8, 8, BATCH), jnp.int32),
    grid=(SEQ // 8,),
    in_specs=[pl.BlockSpec((8, BATCH), lambda k: (k, 0))],
    out_specs=pl.BlockSpec((1, 8, BATCH), lambda k: (k, 0, 0)),
)


def _emb_body(table_hbm, idx_hbm, out_hbm, idx_v, rows_v, trans_v,
              gsem0, gsem1, wsem0, wsem1):
    wid = lax.axis_index("s") * NC + lax.axis_index("c")
    g_base = wid * G_PER_W

    s0 = g_base // NBB
    for k in range(8):
        sk = jnp.minimum(s0 + k, SEQ - 1)
        pltpu.sync_copy(idx_hbm.at[sk // 8, sk % 8], idx_v.at[k])

    gsems = (gsem0, gsem1)
    wsems = (wsem0, wsem1)

    iotas = [lax.iota(jnp.int32, 16) + (16 * j) for j in range(8)]

    def pair_body(p, carry):
        gathers = []
        groups = []
        for b in range(NBUF):
            c = p * NBUF + b
            for gg in range(GPC):
                g = g_base + c * GPC + gg
                s = g // NBB
                bb = g % NBB
                groups.append((g, s, bb))
                gathers.append(pltpu.async_copy(
                    table_hbm.at[idx_v.at[s - s0, pl.ds(bb * 128, 128)]],
                    rows_v.at[b, pl.ds(gg * 128, 128), :],
                    gsems[b]))
        writes = []
        for b in range(NBUF):
            for gg in range(GPC):
                gathers[b * GPC + gg].wait()
            for gg in range(GPC):
                src = rows_v.at[b]
                tdst = trans_v.at[b, gg]
                rows16 = [iotas[j] + (gg * 128) for j in range(8)]

                @plsc.parallel_loop(0, EMBED_DIM, unroll=4)
                def _transpose_e(e0):
                    colv = lax.bitwise_and(iotas[0] + e0, 31)
                    elv = lax.bitwise_and(colv, 7)
                    rv = lax.shift_right_logical(colv, 3)
                    for j in range(8):
                        vec = plsc.load_gather(src, [rows16[j], colv])
                        plsc.store_scatter(tdst, [rv, elv, iotas[j]], vec)

                _, s, bb = groups[b * GPC + gg]
                writes.append(pltpu.async_copy(
                    trans_v.at[b, gg],
                    out_hbm.at[s, :, bb],
                    wsems[b]))
        for w in writes:
            w.wait()
        return carry

    lax.fori_loop(0, CHUNKS // NBUF, pair_body, 0)


_gather_call = pl.kernel(
    _emb_body,
    out_type=jax.ShapeDtypeStruct((SEQ, 4, NBB, 8, 128), jnp.float32),
    name="emb_gather",
    mesh=plsc.VectorSubcoreMesh(core_axis_name="c", subcore_axis_name="s"),
    compiler_params=pltpu.CompilerParams(use_tc_tiling_on_sc=False,
                                         needs_layout_passes=False),
    scratch_types=[
        pltpu.VMEM((8, BATCH), jnp.int32),
        pltpu.VMEM((NBUF, CHUNK_ROWS, EMBED_DIM), jnp.float32),
        pltpu.VMEM((NBUF, GPC, 4, 8, 128), jnp.float32),
        pltpu.SemaphoreType.DMA,
        pltpu.SemaphoreType.DMA,
        pltpu.SemaphoreType.DMA,
        pltpu.SemaphoreType.DMA,
    ],
)


def kernel(x, table):
    out5 = _gather_call(table, _x_relayout(x.T.astype(jnp.int32)))
    return out5.transpose(2, 4, 0, 1, 3).reshape(BATCH, SEQ, EMBED_DIM)
